# baseline (speedup 1.0000x reference)
import functools

import jax
import jax.numpy as jnp
from jax import lax
from jax.experimental import pallas as pl
from jax.experimental.pallas import tpu as pltpu
from jax.experimental.pallas import tpu_sc as plsc

_NUM_W = 1000000
_DIM = 64
_NUM_C = 1024
_B = 4096
_T_LEN = 20
_BODY_LEN = 200

_NC, _NS = 2, 16
_NW = _NC * _NS
_SPW = _B // _NW
_PAIRS = _SPW // 2
_TPP = 2 * _T_LEN
_BPP = 2 * _BODY_LEN
_BCH = 80
_NBCH = _BPP // _BCH


def _sc_body(w_hbm, tidx_hbm, bidx_hbm, tsum_hbm, bsum_hbm,
             tidx_v, bidx_v, trows_v, brows_v, tout_v, bout_v, sem):
    wid = lax.axis_index("s") * _NC + lax.axis_index("c")
    sbase = wid * _SPW
    pltpu.sync_copy(tidx_hbm.at[pl.ds(wid * _SPW * _T_LEN, _SPW * _T_LEN)],
                    tidx_v)
    pltpu.sync_copy(bidx_hbm.at[pl.ds(wid * _SPW * _BODY_LEN,
                                      _SPW * _BODY_LEN)], bidx_v)

    def pair_body(j, carry):
        toff = pl.multiple_of(j * _TPP, 8)
        cps = [pltpu.async_copy(w_hbm.at[tidx_v.at[pl.ds(toff, _TPP)]],
                                trows_v, sem)]
        for c in range(_NBCH):
            boff = pl.multiple_of(j * _BPP + c * _BCH, 8)
            cps.append(pltpu.async_copy(
                w_hbm.at[bidx_v.at[pl.ds(boff, _BCH)]],
                brows_v.at[pl.ds(c * _BCH, _BCH)], sem))
        for cp in cps:
            cp.wait()
        for s in range(2):
            tacc = [jnp.zeros((16,), jnp.float32) for _ in range(4)]
            for r in range(_T_LEN):
                for d in range(4):
                    tacc[d] = tacc[d] + trows_v[s * _T_LEN + r,
                                                pl.ds(d * 16, 16)]
            def red8(it, accs):
                a = list(accs)
                for k in range(8):
                    r = s * _BODY_LEN + it * 8 + k
                    for d in range(4):
                        a[d] = a[d] + brows_v[r, pl.ds(d * 16, 16)]
                return tuple(a)
            bacc = lax.fori_loop(
                0, _BODY_LEN // 8, red8,
                tuple(jnp.zeros((16,), jnp.float32) for _ in range(4)))
            row = 2 * j + s
            for d in range(4):
                tout_v[row, pl.ds(d * 16, 16)] = tacc[d]
                bout_v[row, pl.ds(d * 16, 16)] = bacc[d]
        return carry

    lax.fori_loop(0, _PAIRS, pair_body, 0)
    pltpu.sync_copy(tout_v, tsum_hbm.at[pl.ds(sbase, _SPW)])
    pltpu.sync_copy(bout_v, bsum_hbm.at[pl.ds(sbase, _SPW)])


def _sc_pool(W, title2, body2):
    mesh = plsc.VectorSubcoreMesh(core_axis_name="c", subcore_axis_name="s",
                                  num_cores=_NC, num_subcores=_NS)
    f = pl.kernel(
        _sc_body,
        out_type=(jax.ShapeDtypeStruct((_B, _DIM), jnp.float32),
                  jax.ShapeDtypeStruct((_B, _DIM), jnp.float32)),
        mesh=mesh,
        scratch_types=[
            pltpu.VMEM((_SPW * _T_LEN,), jnp.int32),
            pltpu.VMEM((_SPW * _BODY_LEN,), jnp.int32),
            pltpu.VMEM((_TPP, _DIM), jnp.float32),
            pltpu.VMEM((_BPP, _DIM), jnp.float32),
            pltpu.VMEM((_SPW, _DIM), jnp.float32),
            pltpu.VMEM((_SPW, _DIM), jnp.float32),
            pltpu.SemaphoreType.DMA,
        ],
        compiler_params=pltpu.CompilerParams(use_tc_tiling_on_sc=False),
    )
    return f(W, title2, body2)


def _tc_body(nblk, tsum_ref, bsum_ref, tint_ref, bint_ref, c_ref, out_ref,
             acc_rep, acc_rec, acc_d):
    i = pl.program_id(0)

    @pl.when(i == 0)
    def _():
        acc_rep[...] = jnp.zeros_like(acc_rep)
        acc_rec[...] = jnp.zeros_like(acc_rec)
        acc_d[0] = 0.0

    tcnt = jnp.sum((tint_ref[...] > 0).astype(jnp.float32), axis=1,
                   keepdims=True)
    bcnt = jnp.sum((bint_ref[...] > 0).astype(jnp.float32), axis=1,
                   keepdims=True)
    que = 0.3 * tsum_ref[...] / tcnt + 0.7 * bsum_ref[...] / bcnt
    cmat = c_ref[...]
    score = lax.dot_general(que, cmat, (((1,), (1,)), ((), ())),
                            preferred_element_type=jnp.float32)
    m = jnp.max(score, axis=1, keepdims=True)
    e = jnp.exp(score - m)
    probs = e / jnp.sum(e, axis=1, keepdims=True)
    rec = lax.dot_general(probs, cmat, (((1,), (0,)), ((), ())),
                          preferred_element_type=jnp.float32)
    n_rep = jnp.sqrt(jnp.sum(que * que, axis=1, keepdims=True))
    n_rec = jnp.sqrt(jnp.sum(rec * rec, axis=1, keepdims=True))
    denom = jnp.maximum(n_rec * n_rep, 1e-8)
    diag = jnp.sum(rec * que, axis=1, keepdims=True) / denom
    rep_hat = que / jnp.maximum(n_rep, 1e-20)
    rec_hat = rec / jnp.maximum(n_rec, 1e-20)

    acc_rep[...] = acc_rep[...] + jnp.sum(rep_hat, axis=0, keepdims=True)
    acc_rec[...] = acc_rec[...] + jnp.sum(rec_hat, axis=0, keepdims=True)
    acc_d[0] = acc_d[0] + jnp.sum(diag)

    @pl.when(i == nblk - 1)
    def _():
        total = (jnp.float32(_B) * jnp.float32(_B)
                 + jnp.sum(acc_rep[...] * acc_rec[...])
                 - 2.0 * acc_d[0])
        out_ref[...] = jnp.full((1, 1), total, jnp.float32)


def _tc_loss(tsum, bsum, title_int, body_int, C):
    blk = 512
    nblk = _B // blk
    return pl.pallas_call(
        functools.partial(_tc_body, nblk),
        grid=(nblk,),
        in_specs=[
            pl.BlockSpec((blk, _DIM), lambda i: (i, 0)),
            pl.BlockSpec((blk, _DIM), lambda i: (i, 0)),
            pl.BlockSpec((blk, _T_LEN), lambda i: (i, 0)),
            pl.BlockSpec((blk, _BODY_LEN), lambda i: (i, 0)),
            pl.BlockSpec((_NUM_C, _DIM), lambda i: (0, 0)),
        ],
        out_specs=pl.BlockSpec((1, 1), lambda i: (0, 0)),
        out_shape=jax.ShapeDtypeStruct((1, 1), jnp.float32),
        scratch_shapes=[
            pltpu.VMEM((1, _DIM), jnp.float32),
            pltpu.VMEM((1, _DIM), jnp.float32),
            pltpu.SMEM((1,), jnp.float32),
        ],
        compiler_params=pltpu.CompilerParams(
            dimension_semantics=("arbitrary",)),
    )(tsum, bsum, title_int, body_int, C)


def kernel(title_int, body_int, user_int, W, C):
    title2 = title_int.reshape(_B * _T_LEN).astype(jnp.int32)
    body2 = body_int.reshape(_B * _BODY_LEN).astype(jnp.int32)
    tsum, bsum = _sc_pool(W, title2, body2)
    out = _tc_loss(tsum, bsum, title_int.astype(jnp.int32),
                   body_int.astype(jnp.int32), C)
    return out[0, 0]
